# baseline (speedup 1.0000x reference)
import functools
import jax
import jax.numpy as jnp
from jax import lax
from jax.experimental import pallas as pl
from jax.experimental.pallas import tpu as pltpu
from jax.experimental.pallas import tpu_sc as plsc

NBUF = 4


def kernel(x, pe):
    B, L, D = x.shape
    R = B * L
    NC, NS = 2, 16
    NW = NC * NS
    RWL = L // NW
    C = 8
    NCH = RWL // C
    CW = C * D
    T = NCH * B
    UNROLL = 8

    mesh = plsc.VectorSubcoreMesh(core_axis_name="c", subcore_axis_name="s")

    @functools.partial(
        pl.kernel, mesh=mesh,
        out_type=jax.ShapeDtypeStruct((R, D), jnp.float32),
        scratch_types=(
            [pltpu.VMEM((C, D), jnp.float32) for _ in range(NBUF)]
            + [pltpu.VMEM((C, D), jnp.float32) for _ in range(2)]
            + [pltpu.SemaphoreType.DMA for _ in range(NBUF)]
            + [pltpu.SemaphoreType.DMA for _ in range(2)]
            + [pltpu.SemaphoreType.DMA for _ in range(NBUF)]
        ),
        compiler_params=pltpu.CompilerParams(use_tc_tiling_on_sc=True),
    )
    def sc_add(x_hbm, pe_hbm, out_hbm, *scratch):
        xbufs = scratch[0:NBUF]
        pbufs = scratch[NBUF:NBUF + 2]
        xsems = scratch[NBUF + 2:2 * NBUF + 2]
        psems = scratch[2 * NBUF + 2:2 * NBUF + 4]
        osems = scratch[2 * NBUF + 4:3 * NBUF + 4]

        w = lax.axis_index("c") * NS + lax.axis_index("s")
        lbase = w * RWL

        def x_copy(slot, c, b):
            rows = b * L + lbase + c * C
            return pltpu.make_async_copy(
                x_hbm.at[pl.ds(rows, C), :], xbufs[slot], xsems[slot])

        def pe_copy(par, c):
            return pltpu.make_async_copy(
                pe_hbm.at[pl.ds(lbase + c * C, C), :], pbufs[par], psems[par])

        def out_copy(slot, c, b):
            rows = b * L + lbase + c * C
            return pltpu.make_async_copy(
                xbufs[slot], out_hbm.at[pl.ds(rows, C), :], osems[slot])

        def compute(slot, par):
            xbuf, pbuf = xbufs[slot], pbufs[par]

            def col_body(k, carry):
                s = pl.ds(k * 16, 16)
                for r in range(C):
                    xbuf[r, s] = xbuf[r, s] + pbuf[r, s]
                return carry

            lax.fori_loop(0, D // 16, col_body, 0)

        pe_copy(0, 0).start()
        x_copy(0, 0, 0).start()

        def outer(s, carry):
            c0 = s * 2
            for j in range(UNROLL):
                slot = j % NBUF
                b = j % B
                cj = j // B
                c = c0 + cj
                gt = s * UNROLL + j

                nslot = (j + 1) % NBUF
                nb = (j + 1) % B
                ncc = c0 + (j + 1) // B

                @pl.when(jnp.logical_or(s > 0, j >= NBUF - 1))
                def _():
                    out_copy(nslot, 0, 0).wait()

                @pl.when(gt + 1 < T)
                def _():
                    x_copy(nslot, ncc, nb).start()

                if b == 0:
                    npar = (cj + 1) % 2

                    @pl.when(c + 1 < NCH)
                    def _():
                        pe_copy(npar, c + 1).start()

                x_copy(slot, c, b).wait()
                if b == 0:
                    pe_copy(cj, c).wait()

                compute(slot, cj)
                out_copy(slot, c, b).start()
            return carry

        lax.fori_loop(0, NCH // 2, outer, 0)

        for k in range(T - NBUF + 1, T):
            out_copy(k % NBUF, 0, 0).wait()

    out = sc_add(x.reshape(R, D), pe)
    return out.reshape(B, L, D)


# speedup vs baseline: 1.0419x; 1.0419x over previous
import functools
import jax
import jax.numpy as jnp
from jax import lax
from jax.experimental import pallas as pl
from jax.experimental.pallas import tpu as pltpu
from jax.experimental.pallas import tpu_sc as plsc

NBUF = 4


def kernel(x, pe):
    B, L, D = x.shape
    R = B * L
    NC, NS = 2, 16
    NW = NC * NS
    RWL = L // NW
    C = 16
    NCH = RWL // C
    CW = C * D
    T = NCH * B
    UNROLL = 8

    mesh = plsc.VectorSubcoreMesh(core_axis_name="c", subcore_axis_name="s")

    @functools.partial(
        pl.kernel, mesh=mesh,
        out_type=jax.ShapeDtypeStruct((R, D), jnp.float32),
        scratch_types=(
            [pltpu.VMEM((C, D), jnp.float32) for _ in range(NBUF)]
            + [pltpu.VMEM((C, D), jnp.float32) for _ in range(2)]
            + [pltpu.SemaphoreType.DMA for _ in range(NBUF)]
            + [pltpu.SemaphoreType.DMA for _ in range(2)]
            + [pltpu.SemaphoreType.DMA for _ in range(NBUF)]
        ),
        compiler_params=pltpu.CompilerParams(use_tc_tiling_on_sc=True),
    )
    def sc_add(x_hbm, pe_hbm, out_hbm, *scratch):
        xbufs = scratch[0:NBUF]
        pbufs = scratch[NBUF:NBUF + 2]
        xsems = scratch[NBUF + 2:2 * NBUF + 2]
        psems = scratch[2 * NBUF + 2:2 * NBUF + 4]
        osems = scratch[2 * NBUF + 4:3 * NBUF + 4]

        w = lax.axis_index("c") * NS + lax.axis_index("s")
        lbase = w * RWL

        def x_copy(slot, c, b):
            rows = b * L + lbase + c * C
            return pltpu.make_async_copy(
                x_hbm.at[pl.ds(rows, C), :], xbufs[slot], xsems[slot])

        def pe_copy(par, c):
            return pltpu.make_async_copy(
                pe_hbm.at[pl.ds(lbase + c * C, C), :], pbufs[par], psems[par])

        H = C // 2

        def out_half(slot, c, b, h):
            rows = b * L + lbase + c * C + h * H
            return pltpu.make_async_copy(
                xbufs[slot].at[pl.ds(h * H, H), :],
                out_hbm.at[pl.ds(rows, H), :], osems[slot])

        def compute_half(slot, par, h):
            xbuf, pbuf = xbufs[slot], pbufs[par]

            def col_body(k, carry):
                s = pl.ds(k * 16, 16)
                for r in range(h * H, (h + 1) * H):
                    xbuf[r, s] = xbuf[r, s] + pbuf[r, s]
                return carry

            lax.fori_loop(0, D // 16, col_body, 0)

        pe_copy(0, 0).start()
        x_copy(0, 0, 0).start()

        def outer(s, carry):
            c0 = s * 2
            for j in range(UNROLL):
                slot = j % NBUF
                b = j % B
                cj = j // B
                c = c0 + cj
                gt = s * UNROLL + j

                nslot = (j + 1) % NBUF
                nb = (j + 1) % B
                ncc = c0 + (j + 1) // B

                @pl.when(jnp.logical_or(s > 0, j >= NBUF - 1))
                def _():
                    out_half(nslot, 0, 0, 0).wait()
                    out_half(nslot, 0, 0, 1).wait()

                @pl.when(gt + 1 < T)
                def _():
                    x_copy(nslot, ncc, nb).start()

                if b == 0:
                    npar = (cj + 1) % 2

                    @pl.when(c + 1 < NCH)
                    def _():
                        pe_copy(npar, c + 1).start()

                x_copy(slot, c, b).wait()
                if b == 0:
                    pe_copy(cj, c).wait()

                compute_half(slot, cj, 0)
                out_half(slot, c, b, 0).start()
                compute_half(slot, cj, 1)
                out_half(slot, c, b, 1).start()
            return carry

        lax.fori_loop(0, NCH // 2, outer, 0)

        for k in range(T - NBUF + 1, T):
            out_half(k % NBUF, 0, 0, 0).wait()
            out_half(k % NBUF, 0, 0, 1).wait()

    out = sc_add(x.reshape(R, D), pe)
    return out.reshape(B, L, D)


# speedup vs baseline: 1.1390x; 1.0932x over previous
import functools
import jax
import jax.numpy as jnp
from jax import lax
from jax.experimental import pallas as pl
from jax.experimental.pallas import tpu as pltpu
from jax.experimental.pallas import tpu_sc as plsc

NBUF = 4


def kernel(x, pe):
    B, L, D = x.shape
    R = B * L
    NC, NS = 2, 16
    NW = NC * NS
    RWL = L // NW
    C = 16
    NCH = RWL // C
    CW = C * D
    T = NCH * B
    UNROLL = 8

    mesh = plsc.VectorSubcoreMesh(core_axis_name="c", subcore_axis_name="s")

    @functools.partial(
        pl.kernel, mesh=mesh,
        out_type=jax.ShapeDtypeStruct((R, D), jnp.float32),
        scratch_types=(
            [pltpu.VMEM((C, D), jnp.float32) for _ in range(NBUF)]
            + [pltpu.VMEM((C, D), jnp.float32) for _ in range(2)]
            + [pltpu.SemaphoreType.DMA for _ in range(NBUF)]
            + [pltpu.SemaphoreType.DMA for _ in range(2)]
            + [pltpu.SemaphoreType.DMA for _ in range(NBUF)]
        ),
        compiler_params=pltpu.CompilerParams(use_tc_tiling_on_sc=True),
    )
    def sc_add(x_hbm, pe_hbm, out_hbm, *scratch):
        xbufs = scratch[0:NBUF]
        pbufs = scratch[NBUF:NBUF + 2]
        xsems = scratch[NBUF + 2:2 * NBUF + 2]
        psems = scratch[2 * NBUF + 2:2 * NBUF + 4]
        osems = scratch[2 * NBUF + 4:3 * NBUF + 4]

        w = lax.axis_index("c") * NS + lax.axis_index("s")
        lbase = w * RWL

        def x_copy(slot, c, b):
            rows = b * L + lbase + c * C
            return pltpu.make_async_copy(
                x_hbm.at[pl.ds(rows, C), :], xbufs[slot], xsems[slot])

        def pe_copy(par, c):
            return pltpu.make_async_copy(
                pe_hbm.at[pl.ds(lbase + c * C, C), :], pbufs[par], psems[par])

        def out_copy(slot, c, b):
            rows = b * L + lbase + c * C
            return pltpu.make_async_copy(
                xbufs[slot], out_hbm.at[pl.ds(rows, C), :], osems[slot])

        def compute(slot, par):
            xbuf, pbuf = xbufs[slot], pbufs[par]

            def col_body(k, carry):
                s = pl.ds(k * 16, 16)
                for r in range(C):
                    xbuf[r, s] = xbuf[r, s] + pbuf[r, s]
                return carry

            lax.fori_loop(0, D // 16, col_body, 0)

        pe_copy(0, 0).start()
        x_copy(0, 0, 0).start()

        def outer(s, carry):
            c0 = s * 2
            for j in range(UNROLL):
                slot = j % NBUF
                b = j % B
                cj = j // B
                c = c0 + cj
                gt = s * UNROLL + j

                nslot = (j + 1) % NBUF
                nb = (j + 1) % B
                ncc = c0 + (j + 1) // B

                @pl.when(jnp.logical_or(s > 0, j >= NBUF - 1))
                def _():
                    out_copy(nslot, 0, 0).wait()

                @pl.when(gt + 1 < T)
                def _():
                    x_copy(nslot, ncc, nb).start()

                if b == 0:
                    npar = (cj + 1) % 2

                    @pl.when(c + 1 < NCH)
                    def _():
                        pe_copy(npar, c + 1).start()

                x_copy(slot, c, b).wait()
                if b == 0:
                    pe_copy(cj, c).wait()

                compute(slot, cj)
                out_copy(slot, c, b).start()
            return carry

        lax.fori_loop(0, NCH // 2, outer, 0)

        for k in range(T - NBUF + 1, T):
            out_copy(k % NBUF, 0, 0).wait()

    out = sc_add(x.reshape(R, D), pe)
    return out.reshape(B, L, D)


# speedup vs baseline: 1.1512x; 1.0106x over previous
import functools
import jax
import jax.numpy as jnp
from jax import lax
from jax.experimental import pallas as pl
from jax.experimental.pallas import tpu as pltpu
from jax.experimental.pallas import tpu_sc as plsc

NBUF = 4


def kernel(x, pe):
    B, L, D = x.shape
    R = B * L
    NC, NS = 2, 16
    NW = NC * NS
    RWL = L // NW
    C = 16
    NCH = RWL // C
    CW = C * D
    T = NCH * B
    UNROLL = 8

    mesh = plsc.VectorSubcoreMesh(core_axis_name="c", subcore_axis_name="s")

    @functools.partial(
        pl.kernel, mesh=mesh,
        out_type=jax.ShapeDtypeStruct((R, D), jnp.float32),
        scratch_types=(
            [pltpu.VMEM((C, D), jnp.float32) for _ in range(NBUF)]
            + [pltpu.VMEM((C, D), jnp.float32) for _ in range(2)]
            + [pltpu.SemaphoreType.DMA for _ in range(NBUF)]
            + [pltpu.SemaphoreType.DMA for _ in range(2)]
            + [pltpu.SemaphoreType.DMA for _ in range(NBUF)]
        ),
        compiler_params=pltpu.CompilerParams(use_tc_tiling_on_sc=True),
    )
    def sc_add(x_hbm, pe_hbm, out_hbm, *scratch):
        xbufs = scratch[0:NBUF]
        pbufs = scratch[NBUF:NBUF + 2]
        xsems = scratch[NBUF + 2:2 * NBUF + 2]
        psems = scratch[2 * NBUF + 2:2 * NBUF + 4]
        osems = scratch[2 * NBUF + 4:3 * NBUF + 4]

        w = lax.axis_index("c") * NS + lax.axis_index("s")
        lbase = w * RWL

        def x_copy(slot, c, b):
            rows = b * L + lbase + c * C
            return pltpu.make_async_copy(
                x_hbm.at[pl.ds(rows, C), :], xbufs[slot], xsems[slot])

        def pe_copy(par, c):
            return pltpu.make_async_copy(
                pe_hbm.at[pl.ds(lbase + c * C, C), :], pbufs[par], psems[par])

        def out_copy(slot, c, b):
            rows = b * L + lbase + c * C
            return pltpu.make_async_copy(
                xbufs[slot], out_hbm.at[pl.ds(rows, C), :], osems[slot])

        def compute(slot, par):
            xbuf, pbuf = xbufs[slot], pbufs[par]

            def col_body(k, carry):
                s = pl.ds(k * 16, 16)
                for r in range(C):
                    xbuf[r, s] = xbuf[r, s] + pbuf[r, s]
                return carry

            lax.fori_loop(0, D // 16, col_body, 0)

        pe_copy(0, 0).start()
        x_copy(0, 0, 0).start()
        x_copy(1, 0, 1).start()

        def outer(s, carry):
            c0 = s * 2
            for j in range(UNROLL):
                slot = j % NBUF
                b = j % B
                cj = j // B
                c = c0 + cj
                gt = s * UNROLL + j

                nslot = (j + 2) % NBUF
                nb = (j + 2) % B
                ncc = c0 + (j + 2) // B

                @pl.when(jnp.logical_or(s > 0, j >= NBUF - 2))
                def _():
                    out_copy(nslot, 0, 0).wait()

                @pl.when(gt + 2 < T)
                def _():
                    x_copy(nslot, ncc, nb).start()

                if b == 0:
                    npar = (cj + 1) % 2

                    @pl.when(c + 1 < NCH)
                    def _():
                        pe_copy(npar, c + 1).start()

                x_copy(slot, c, b).wait()
                if b == 0:
                    pe_copy(cj, c).wait()

                compute(slot, cj)
                out_copy(slot, c, b).start()
            return carry

        lax.fori_loop(0, NCH // 2, outer, 0)

        for k in range(T - NBUF + 2, T):
            out_copy(k % NBUF, 0, 0).wait()

    out = sc_add(x.reshape(R, D), pe)
    return out.reshape(B, L, D)
